# baseline (speedup 1.0000x reference)
import jax
import jax.numpy as jnp
from jax.experimental import pallas as pl
from jax.experimental.pallas import tpu as pltpu

_TN = 512
_TF = 512


def _mlp_kernel(x_ref, wg_hbm, wu_hbm, wd_hbm, o_ref,
                wgs, wus, wds, wgb, wub, wdb, ab,
                swg, swu, swd):
    d = x_ref.shape[1]
    ff = wgs.shape[1]
    nf = ff // _TF
    i = pl.program_id(0)

    cwg = [pltpu.make_async_copy(wg_hbm.at[:, pl.ds(f * _TF, _TF)],
                                 wgs.at[:, pl.ds(f * _TF, _TF)], swg.at[f])
           for f in range(nf)]
    cwu = [pltpu.make_async_copy(wu_hbm.at[:, pl.ds(f * _TF, _TF)],
                                 wus.at[:, pl.ds(f * _TF, _TF)], swu.at[f])
           for f in range(nf)]
    cwd = pltpu.make_async_copy(wd_hbm, wds, swd)

    @pl.when(i == 0)
    def _issue():
        for f in range(nf):
            cwg[f].start()
            cwu[f].start()
        cwd.start()

    xi = x_ref[...].astype(jnp.bfloat16)
    for f in range(nf):
        @pl.when(i == 0)
        def _land(f=f):
            cwg[f].wait()
            cwu[f].wait()
            sl = pl.ds(f * _TF, _TF)
            wgb[:, sl] = wgs[:, sl].astype(jnp.bfloat16)
            wub[:, sl] = wus[:, sl].astype(jnp.bfloat16)
        sl = pl.ds(f * _TF, _TF)
        g = jnp.dot(xi, wgb[:, sl], preferred_element_type=jnp.float32)
        u = jnp.dot(xi, wub[:, sl], preferred_element_type=jnp.float32)
        a = (g * jax.nn.sigmoid(g)) * u
        ab[:, sl] = a.astype(jnp.bfloat16)

    @pl.when(i == 0)
    def _land_wd():
        cwd.wait()
        wdb[...] = wds[...].astype(jnp.bfloat16)

    o_ref[...] = jnp.dot(ab[...], wdb[...], preferred_element_type=jnp.float32)


@jax.jit
def kernel(hidden_states, router_w, w_gate_proj, w_up_proj, w_down_proj):
    b, s, d = hidden_states.shape
    n = b * s
    ff = w_gate_proj.shape[1]
    x = hidden_states.reshape(n, d)
    hbm = pl.BlockSpec(memory_space=pltpu.MemorySpace.HBM)
    out = pl.pallas_call(
        _mlp_kernel,
        grid=(n // _TN,),
        in_specs=[
            pl.BlockSpec((_TN, d), lambda i: (i, 0)),
            hbm, hbm, hbm,
        ],
        out_specs=pl.BlockSpec((_TN, d), lambda i: (i, 0)),
        out_shape=jax.ShapeDtypeStruct((n, d), jnp.float32),
        scratch_shapes=[
            pltpu.VMEM((d, ff), jnp.float32),
            pltpu.VMEM((d, ff), jnp.float32),
            pltpu.VMEM((ff, d), jnp.float32),
            pltpu.VMEM((d, ff), jnp.bfloat16),
            pltpu.VMEM((d, ff), jnp.bfloat16),
            pltpu.VMEM((ff, d), jnp.bfloat16),
            pltpu.VMEM((_TN, ff), jnp.bfloat16),
            pltpu.SemaphoreType.DMA((ff // _TF,)),
            pltpu.SemaphoreType.DMA((ff // _TF,)),
            pltpu.SemaphoreType.DMA,
        ],
    )(x, w_gate_proj, w_up_proj, w_down_proj)
    return out.reshape(b, s, d)


# speedup vs baseline: 1.3721x; 1.3721x over previous
import jax
import jax.numpy as jnp
from jax.experimental import pallas as pl
from jax.experimental.pallas import tpu as pltpu

_TN = 512
_TF = 512


def _mlp_kernel(x_hbm, wg_hbm, wu_hbm, wd_hbm, o_hbm,
                xs, wgs, wus, wds, ab,
                sx, swg, swu, swd, so):
    n, d = xs.shape
    ff = wgs.shape[1]
    ni, nf = n // _TN, ff // _TF

    cx = [pltpu.make_async_copy(x_hbm.at[pl.ds(i * _TN, _TN), :],
                                xs.at[pl.ds(i * _TN, _TN), :], sx.at[i])
          for i in range(ni)]
    cwg = [pltpu.make_async_copy(wg_hbm.at[:, pl.ds(f * _TF, _TF)],
                                 wgs.at[:, pl.ds(f * _TF, _TF)], swg.at[f])
           for f in range(nf)]
    cwu = [pltpu.make_async_copy(wu_hbm.at[:, pl.ds(f * _TF, _TF)],
                                 wus.at[:, pl.ds(f * _TF, _TF)], swu.at[f])
           for f in range(nf)]
    cwd = pltpu.make_async_copy(wd_hbm, wds, swd)

    cx[0].start()
    cwg[0].start()
    cwu[0].start()
    for i in range(1, ni):
        cx[i].start()
    for f in range(1, nf):
        cwg[f].start()
        cwu[f].start()
    cwd.start()

    for f in range(nf):
        cwg[f].wait()
        cwu[f].wait()
        wgf = wgs[:, f * _TF:(f + 1) * _TF]
        wuf = wus[:, f * _TF:(f + 1) * _TF]
        for i in range(ni):
            if f == 0:
                cx[i].wait()
            xi = xs[pl.ds(i * _TN, _TN), :]
            g = jnp.dot(xi, wgf, preferred_element_type=jnp.float32)
            u = jnp.dot(xi, wuf, preferred_element_type=jnp.float32)
            a = (g * jax.nn.sigmoid(g)) * u
            ab[pl.ds(i * _TN, _TN), pl.ds(f * _TF, _TF)] = a

    cwd.wait()
    co = [pltpu.make_async_copy(xs.at[pl.ds(i * _TN, _TN), :],
                                o_hbm.at[pl.ds(i * _TN, _TN), :], so.at[i])
          for i in range(ni)]
    for i in range(ni):
        xs[pl.ds(i * _TN, _TN), :] = jnp.dot(
            ab[pl.ds(i * _TN, _TN), :], wds[...],
            preferred_element_type=jnp.float32)
        co[i].start()
    for i in range(ni):
        co[i].wait()


@jax.jit
def kernel(hidden_states, router_w, w_gate_proj, w_up_proj, w_down_proj):
    b, s, d = hidden_states.shape
    n = b * s
    ff = w_gate_proj.shape[1]
    x = hidden_states.reshape(n, d)
    hbm = pl.BlockSpec(memory_space=pltpu.MemorySpace.HBM)
    out = pl.pallas_call(
        _mlp_kernel,
        in_specs=[hbm, hbm, hbm, hbm],
        out_specs=hbm,
        out_shape=jax.ShapeDtypeStruct((n, d), jnp.float32),
        scratch_shapes=[
            pltpu.VMEM((n, d), jnp.float32),
            pltpu.VMEM((d, ff), jnp.float32),
            pltpu.VMEM((d, ff), jnp.float32),
            pltpu.VMEM((ff, d), jnp.float32),
            pltpu.VMEM((n, ff), jnp.float32),
            pltpu.SemaphoreType.DMA((n // _TN,)),
            pltpu.SemaphoreType.DMA((ff // _TF,)),
            pltpu.SemaphoreType.DMA((ff // _TF,)),
            pltpu.SemaphoreType.DMA,
            pltpu.SemaphoreType.DMA((n // _TN,)),
        ],
    )(x, w_gate_proj, w_up_proj, w_down_proj)
    return out.reshape(b, s, d)
